# baseline (speedup 1.0000x reference)
import functools

import jax
import jax.numpy as jnp
from jax import lax
from jax.experimental import pallas as pl
from jax.experimental.pallas import tpu as pltpu
from jax.experimental.pallas import tpu_sc as plsc

N_NODES = 100000
D_FEAT = 128
NSEG = 512

NC = 2
NS = 16
NW = NC * NS
RPW = N_NODES // NW
CH = 125
NCHUNK = RPW // CH
STG = 16
DUMMY = NSEG
IDS_LOAD = RPW + 8 + 16
IDS_PAD = 100096
SEG_PW = NSEG // NS


def _seg_sum_body(feat_hbm, ids_hbm, zeros_hbm, out_hbm,
                  ids_v, fb0, fb1, stg, shared, sem0, sem1):
    c = lax.axis_index("c")
    s = lax.axis_index("s")
    gwid = s * NC + c
    base = gwid * RPW
    base_al = (base // 8) * 8
    shift = base - base_al

    pltpu.sync_copy(zeros_hbm.at[pl.ds(s * SEG_PW, SEG_PW)],
                    shared.at[pl.ds(s * SEG_PW, SEG_PW)])
    plsc.subcore_barrier()

    pltpu.sync_copy(ids_hbm.at[pl.ds(base_al, IDS_LOAD)], ids_v)

    bufs = [fb0, fb1]
    sems = [sem0, sem1]
    copies = [None, None]
    fbase = base * D_FEAT
    copies[0] = pltpu.async_copy(feat_hbm.at[pl.ds(fbase, CH * D_FEAT)],
                                 fb0, sem0)

    zero16 = jnp.zeros((16,), jnp.float32)
    dummy_ids = jnp.full((STG,), DUMMY, jnp.int32)
    lanes = lax.iota(jnp.int32, 16)

    def stage_acc(k, acc):
        row_ref = stg.at[k]
        for j in range(8):
            row_ref[pl.ds(16 * j, 16)] = acc[j]

    carry = (ids_v[pl.ds(shift, 16)][0], jnp.int32(0), dummy_ids) + (zero16,) * 8

    for ci in range(NCHUNK):
        b = ci & 1
        if ci + 1 < NCHUNK:
            copies[1 - b] = pltpu.async_copy(
                feat_hbm.at[pl.ds(fbase + (ci + 1) * CH * D_FEAT, CH * D_FEAT)],
                bufs[1 - b], sems[1 - b])
        copies[b].wait()
        fb = bufs[b]

        def body(r, carry, fb=fb, ci=ci):
            cur, k, sid = carry[0], carry[1], carry[2]
            acc = carry[3:]
            rid = ids_v[pl.ds(shift + ci * CH + r, 16)][0]
            row = [fb[pl.ds(r * D_FEAT + 16 * j, 16)] for j in range(8)]
            new = rid != cur

            @pl.when(new)
            def _stage():
                stage_acc(k, acc)

            kk = jnp.where(new, k, jnp.int32(-1))
            sid2 = jnp.where(lanes == kk, cur, sid)
            k2 = jnp.where(new, k + 1, k)

            @pl.when(k2 == STG)
            def _flush():
                pltpu.sync_copy(stg, shared.at[sid2], add=True)

            fl = jnp.where(k2 == STG, jnp.int32(1), jnp.int32(0))
            k3 = k2 * (1 - fl)
            sid3 = sid2 * (1 - fl) + dummy_ids * fl
            keep = jnp.where(new, jnp.float32(0), jnp.float32(1))
            acc2 = tuple(row[j] + acc[j] * keep for j in range(8))
            return (rid, k3, sid3) + acc2

        carry = lax.fori_loop(0, CH, body, carry)

    cur, k, sid = carry[0], carry[1], carry[2]
    acc = carry[3:]
    stage_acc(k, acc)
    sid = jnp.where(lanes == k, cur, sid)
    pltpu.sync_copy(stg, shared.at[sid], add=True)

    plsc.subcore_barrier()
    pltpu.sync_copy(shared.at[pl.ds(s * SEG_PW, SEG_PW)],
                    out_hbm.at[pl.ds(c * NSEG + s * SEG_PW, SEG_PW)])


_seg_sum = functools.partial(
    pl.kernel,
    out_type=jax.ShapeDtypeStruct((NC * NSEG, D_FEAT), jnp.float32),
    mesh=plsc.VectorSubcoreMesh(core_axis_name="c", subcore_axis_name="s"),
    scratch_types=[
        pltpu.VMEM((IDS_LOAD,), jnp.int32),
        pltpu.VMEM((CH * D_FEAT,), jnp.float32),
        pltpu.VMEM((CH * D_FEAT,), jnp.float32),
        pltpu.VMEM((STG, D_FEAT), jnp.float32),
        pltpu.VMEM_SHARED((NSEG + 8, D_FEAT), jnp.float32),
        pltpu.SemaphoreType.DMA,
        pltpu.SemaphoreType.DMA,
    ],
)(_seg_sum_body)


def _add_halves_body(p_ref, o_ref):
    o_ref[...] = p_ref[0] + p_ref[1]


_add_halves = pl.pallas_call(
    _add_halves_body,
    out_shape=jax.ShapeDtypeStruct((NSEG, D_FEAT), jnp.float32),
)


def kernel(feat, segment_ids):
    ids = segment_ids.astype(jnp.int32)
    ids = jnp.pad(ids, (0, IDS_PAD - N_NODES))
    zeros = jnp.zeros((NSEG, D_FEAT), jnp.float32)
    partial = _seg_sum(feat.reshape(-1), ids, zeros)
    return _add_halves(partial.reshape(NC, NSEG, D_FEAT))


# speedup vs baseline: 1.3199x; 1.3199x over previous
import functools

import jax
import jax.numpy as jnp
from jax import lax
from jax.experimental import pallas as pl
from jax.experimental.pallas import tpu as pltpu
from jax.experimental.pallas import tpu_sc as plsc

N_NODES = 100000
D_FEAT = 128
NSEG = 512

NC = 2
NS = 16
NW = NC * NS
RPW = N_NODES // NW
CHUNK_ROWS = (128,) * 24 + (53,)
CHMAX = 128
STG = 16
DUMMY = NSEG
IDS_LOAD = RPW + 8 + 16
IDS_PAD = 100096
SEG_PW = NSEG // NS


def _seg_sum_body(feat_hbm, ids_hbm, zeros_hbm, out_hbm,
                  ids_v, fb0, fb1, stg, kv, sidv_ref, shared, sem0, sem1):
    c = lax.axis_index("c")
    s = lax.axis_index("s")
    gwid = s * NC + c
    base = gwid * RPW
    base_al = (base // 8) * 8
    shift = base - base_al

    pltpu.sync_copy(zeros_hbm.at[pl.ds(s * SEG_PW, SEG_PW)],
                    shared.at[pl.ds(s * SEG_PW, SEG_PW)])
    plsc.subcore_barrier()

    pltpu.sync_copy(ids_hbm.at[pl.ds(base_al, IDS_LOAD)], ids_v)

    bufs = [fb0, fb1]
    sems = [sem0, sem1]
    copies = [None, None]
    fbase = base * D_FEAT
    copies[0] = pltpu.async_copy(
        feat_hbm.at[pl.ds(fbase, CHUNK_ROWS[0] * D_FEAT)],
        fb0.at[pl.ds(0, CHUNK_ROWS[0] * D_FEAT)], sem0)

    zero16 = jnp.zeros((16,), jnp.float32)
    zero16i = jnp.zeros((16,), jnp.int32)
    dummy_ids = jnp.full((STG,), DUMMY, jnp.int32)
    lanes = lax.iota(jnp.int32, 16)

    kv[...] = zero16i
    sidv_ref[...] = dummy_ids

    def load_row(fb, roff):
        return tuple(fb[pl.ds(roff * D_FEAT + 16 * j, 16)] for j in range(8))

    def stage(cur, acc):
        k = kv[pl.ds(0, 16)][0]
        row_ref = stg.at[k]
        for j in range(8):
            row_ref[pl.ds(16 * j, 16)] = acc[j]
        sidv = jnp.where(lanes == k, cur, sidv_ref[pl.ds(0, 16)])
        k2 = k + 1

        @pl.when(k2 == STG)
        def _flush():
            pltpu.sync_copy(stg, shared.at[sidv], add=True)

        fl = jnp.where(k2 == STG, jnp.int32(1), jnp.int32(0))
        kv[...] = zero16i + k2 * (1 - fl)
        sidv_ref[...] = sidv * (1 - fl) + dummy_ids * fl

    def row_step(cur, acc, rid, row):
        new = rid != cur

        @pl.when(new)
        def _s():
            stage(cur, acc)

        keep = jnp.where(new, jnp.float32(0), jnp.float32(1))
        acc2 = tuple(row[j] + acc[j] * keep for j in range(8))
        return (rid,) + acc2

    carry = (ids_v[pl.ds(shift, 16)][0],) + (zero16,) * 8

    row_off = 0
    for ci, nrows in enumerate(CHUNK_ROWS):
        b = ci & 1
        if ci + 1 < len(CHUNK_ROWS):
            nxt = CHUNK_ROWS[ci + 1]
            copies[1 - b] = pltpu.async_copy(
                feat_hbm.at[pl.ds(fbase + (row_off + nrows) * D_FEAT,
                                  nxt * D_FEAT)],
                bufs[1 - b].at[pl.ds(0, nxt * D_FEAT)], sems[1 - b])
        copies[b].wait()
        fb = bufs[b]
        ngroups = nrows // 16
        ntail = nrows % 16

        def group_body(g, carry, fb=fb, row_off=row_off):
            cur = carry[0]
            acc = carry[1:]
            pos = shift + row_off + g * 16
            iv = ids_v[pl.ds(pos, 16)]
            i0 = iv[0]
            i15 = iv[15]
            same = (jnp.where(i0 == cur, jnp.int32(1), jnp.int32(0))
                    * jnp.where(i15 == i0, jnp.int32(1), jnp.int32(0)))

            @pl.when(same == 0)
            def _rare():
                def rb(i, cin):
                    rid = ids_v[pl.ds(pos + i, 16)][0]
                    row = load_row(fb, g * 16 + i)
                    return row_step(cin[0], cin[1:], rid, row)

                cfin = lax.fori_loop(0, 16, rb, (cur,) + acc)
                stage(cfin[0], cfin[1:])

            rows = [load_row(fb, g * 16 + i) for i in range(16)]
            gsum = rows[0]
            for i in range(1, 16):
                gsum = tuple(gsum[j] + rows[i][j] for j in range(8))
            cf = jnp.where(same == 1, jnp.float32(1), jnp.float32(0))
            acc2 = tuple((acc[j] + gsum[j]) * cf for j in range(8))
            return (i15,) + acc2

        carry = lax.fori_loop(0, ngroups, group_body, carry)

        if ntail:
            def tail_body(i, cin, fb=fb, row_off=row_off, ngroups=ngroups):
                pos = shift + row_off + ngroups * 16 + i
                rid = ids_v[pl.ds(pos, 16)][0]
                row = load_row(fb, ngroups * 16 + i)
                return row_step(cin[0], cin[1:], rid, row)

            carry = lax.fori_loop(0, ntail, tail_body, carry)

        row_off += nrows

    stage(carry[0], carry[1:])
    pltpu.sync_copy(stg, shared.at[sidv_ref[pl.ds(0, 16)]], add=True)

    plsc.subcore_barrier()
    pltpu.sync_copy(shared.at[pl.ds(s * SEG_PW, SEG_PW)],
                    out_hbm.at[pl.ds(c * NSEG + s * SEG_PW, SEG_PW)])


_seg_sum = functools.partial(
    pl.kernel,
    out_type=jax.ShapeDtypeStruct((NC * NSEG, D_FEAT), jnp.float32),
    mesh=plsc.VectorSubcoreMesh(core_axis_name="c", subcore_axis_name="s"),
    scratch_types=[
        pltpu.VMEM((IDS_LOAD,), jnp.int32),
        pltpu.VMEM((CHMAX * D_FEAT,), jnp.float32),
        pltpu.VMEM((CHMAX * D_FEAT,), jnp.float32),
        pltpu.VMEM((STG, D_FEAT), jnp.float32),
        pltpu.VMEM((16,), jnp.int32),
        pltpu.VMEM((16,), jnp.int32),
        pltpu.VMEM_SHARED((NSEG + 8, D_FEAT), jnp.float32),
        pltpu.SemaphoreType.DMA,
        pltpu.SemaphoreType.DMA,
    ],
)(_seg_sum_body)


def _add_halves_body(p_ref, o_ref):
    o_ref[...] = p_ref[0] + p_ref[1]


_add_halves = pl.pallas_call(
    _add_halves_body,
    out_shape=jax.ShapeDtypeStruct((NSEG, D_FEAT), jnp.float32),
)


def kernel(feat, segment_ids):
    ids = segment_ids.astype(jnp.int32)
    ids = jnp.pad(ids, (0, IDS_PAD - N_NODES))
    zeros = jnp.zeros((NSEG, D_FEAT), jnp.float32)
    partial = _seg_sum(feat.reshape(-1), ids, zeros)
    return _add_halves(partial.reshape(NC, NSEG, D_FEAT))
